# baseline (speedup 1.0000x reference)
import functools

import jax
import jax.numpy as jnp
from jax import lax
from jax.experimental import pallas as pl
from jax.experimental.pallas import tpu as pltpu
from jax.experimental.pallas import tpu_sc as plsc

B = 16384
NCAT = 26
NCONT = 26
NCOLS = 52
V = 100000
D = 32
OUT_W = NCONT + NCAT * D

NC = 2
NS = 16
NW = NC * NS
RW = B // NW
M = 128
NG = RW // M
GROUP = 128
NGRP = (M * NCAT) // GROUP


def _gather_body(x_hbm, tab_hbm, emb_hbm, x_v, idx_v, rows_v, sem):
    wid = lax.axis_index("s") * NC + lax.axis_index("c")
    iota = lax.iota(jnp.int32, 16)
    pat_a = iota * V
    pat_b = (iota + 10) * V

    for g in range(NG):
        base = wid * RW + g * M

        pltpu.sync_copy(x_hbm.at[pl.ds(base, M)], x_v)

        @pl.loop(0, M)
        def _build(b):
            ca = x_v[b, pl.ds(NCONT, 16)] + pat_a
            cb = x_v[b, pl.ds(NCONT + 10, 16)] + pat_b
            idx_v[pl.ds(b * NCAT, 16)] = ca
            idx_v[pl.ds(b * NCAT + 10, 16)] = cb

        @pl.loop(0, NGRP)
        def _fire(j):
            pltpu.async_copy(
                tab_hbm.at[idx_v.at[pl.ds(j * GROUP, GROUP)]],
                rows_v.at[pl.ds(j * GROUP, GROUP)],
                sem,
            )

        pltpu.make_async_copy(tab_hbm.at[pl.ds(0, M * NCAT)], rows_v, sem).wait()

        pltpu.sync_copy(rows_v, emb_hbm.at[pl.ds(base * NCAT, M * NCAT)])


def _interleave_body(x_ref, emb_ref, out_ref):
    cont = x_ref[:, :NCONT].astype(jnp.float32)
    out_ref[...] = jnp.concatenate([cont, emb_ref[...]], axis=1)


BM = 1024


@jax.jit
def _run(x, tab_flat):
    gather = pl.kernel(
        _gather_body,
        out_type=jax.ShapeDtypeStruct((B * NCAT, D), jnp.float32),
        mesh=plsc.VectorSubcoreMesh(core_axis_name="c", subcore_axis_name="s"),
        compiler_params=pltpu.CompilerParams(use_tc_tiling_on_sc=False),
        scratch_types=[
            pltpu.VMEM((M, NCOLS), jnp.int32),
            pltpu.VMEM((M * NCAT,), jnp.int32),
            pltpu.VMEM((M * NCAT, D), jnp.float32),
            pltpu.SemaphoreType.DMA,
        ],
    )
    emb = gather(x, tab_flat)

    out = pl.pallas_call(
        _interleave_body,
        out_shape=jax.ShapeDtypeStruct((B, OUT_W), jnp.float32),
        grid=(B // BM,),
        in_specs=[
            pl.BlockSpec((BM, NCOLS), lambda i: (i, 0)),
            pl.BlockSpec((BM, NCAT * D), lambda i: (i, 0)),
        ],
        out_specs=pl.BlockSpec((BM, OUT_W), lambda i: (i, 0)),
    )(x, emb.reshape(B, NCAT * D))
    return out


def kernel(x, tables):
    return _run(x, tables.reshape(NCAT * V, D))


# speedup vs baseline: 1.0320x; 1.0320x over previous
import functools

import jax
import jax.numpy as jnp
from jax import lax
from jax.experimental import pallas as pl
from jax.experimental.pallas import tpu as pltpu
from jax.experimental.pallas import tpu_sc as plsc

B = 16384
NCAT = 26
NCONT = 26
NCOLS = 52
V = 100000
D = 32
OUT_W = NCONT + NCAT * D

NC = 2
NS = 16
NW = NC * NS
RW = B // NW
M = 128
NG = RW // M
GROUP = 128
NGRP = (M * NCAT) // GROUP


def _gather_body(x_hbm, tab_hbm, emb_hbm, x_v, idx_v, rows_v, sem):
    wid = lax.axis_index("s") * NC + lax.axis_index("c")
    iota = lax.iota(jnp.int32, 16)
    pat_a = iota * V
    pat_b = (iota + 10) * V

    for g in range(NG):
        base = wid * RW + g * M

        pltpu.sync_copy(x_hbm.at[pl.ds(base, M)], x_v)

        @pl.loop(0, M)
        def _build(b):
            ca = x_v[b, pl.ds(NCONT, 16)] + pat_a
            cb = x_v[b, pl.ds(NCONT + 10, 16)] + pat_b
            idx_v[pl.ds(b * NCAT, 16)] = ca
            idx_v[pl.ds(b * NCAT + 10, 16)] = cb

        @pl.loop(0, NGRP)
        def _fire(j):
            pltpu.async_copy(
                tab_hbm.at[idx_v.at[pl.ds(j * GROUP, GROUP)]],
                rows_v.at[pl.ds(j * GROUP, GROUP)],
                sem,
            )

        pltpu.make_async_copy(tab_hbm.at[pl.ds(0, M * NCAT)], rows_v, sem).wait()

        pltpu.sync_copy(rows_v, emb_hbm.at[pl.ds(base * NCAT, M * NCAT)])


def _interleave_body(x_ref, emb_ref, out_ref):
    cont = x_ref[:, :NCONT].astype(jnp.float32)
    blk = jnp.concatenate([cont, emb_ref[...]], axis=1)
    out_ref[...] = blk.T


BM = 512


@jax.jit
def _run(x, tab_flat):
    gather = pl.kernel(
        _gather_body,
        out_type=jax.ShapeDtypeStruct((B * NCAT, D), jnp.float32),
        mesh=plsc.VectorSubcoreMesh(core_axis_name="c", subcore_axis_name="s"),
        compiler_params=pltpu.CompilerParams(use_tc_tiling_on_sc=False),
        scratch_types=[
            pltpu.VMEM((M, NCOLS), jnp.int32),
            pltpu.VMEM((M * NCAT,), jnp.int32),
            pltpu.VMEM((M * NCAT, D), jnp.float32),
            pltpu.SemaphoreType.DMA,
        ],
    )
    emb = gather(x, tab_flat)

    out_t = pl.pallas_call(
        _interleave_body,
        out_shape=jax.ShapeDtypeStruct((OUT_W, B), jnp.float32),
        grid=(B // BM,),
        in_specs=[
            pl.BlockSpec((BM, NCOLS), lambda i: (i, 0)),
            pl.BlockSpec((BM, NCAT * D), lambda i: (i, 0)),
        ],
        out_specs=pl.BlockSpec((OUT_W, BM), lambda i: (0, i)),
    )(x, emb.reshape(B, NCAT * D))
    return out_t.T


def kernel(x, tables):
    return _run(x, tables.reshape(NCAT * V, D))
